# Initial kernel scaffold; baseline (speedup 1.0000x reference)
import functools

import jax
import jax.numpy as jnp
from jax import lax
from jax.experimental import pallas as pl
from jax.experimental.pallas import tpu as pltpu
from jax.experimental.pallas import tpu_sc as plsc

EMBEDDING_DIM = 64
TOTAL = 16384 * 50
NUM_CORES = 2
NUM_SUBCORES = 16
NUM_WORKERS = NUM_CORES * NUM_SUBCORES
PER_WORKER = TOTAL // NUM_WORKERS
CHUNK = 1024
NUM_CHUNKS = PER_WORKER // CHUNK

_mesh = plsc.VectorSubcoreMesh(core_axis_name="c", subcore_axis_name="s")


@functools.partial(
    pl.kernel,
    mesh=_mesh,
    out_type=jax.ShapeDtypeStruct((TOTAL, EMBEDDING_DIM), jnp.float32),
    scratch_types=[
        pltpu.VMEM((CHUNK,), jnp.int32),
        pltpu.VMEM((CHUNK, EMBEDDING_DIM), jnp.float32),
        pltpu.SemaphoreType.DMA,
    ],
)
def _gather(idx_hbm, table_hbm, out_hbm, idx_v, rows_v, sem):
    wid = lax.axis_index("s") * NUM_CORES + lax.axis_index("c")
    base = wid * PER_WORKER

    def body(c, carry):
        off = base + c * CHUNK
        pltpu.sync_copy(idx_hbm.at[pl.ds(off, CHUNK)], idx_v)
        pltpu.async_copy(table_hbm.at[idx_v], rows_v, sem).wait()
        pltpu.sync_copy(rows_v, out_hbm.at[pl.ds(off, CHUNK)])
        return carry

    lax.fori_loop(0, NUM_CHUNKS, body, 0)


def kernel(token_ids, embeddings):
    flat = token_ids.reshape(-1).astype(jnp.int32)
    out = _gather(flat, embeddings)
    return out.reshape(token_ids.shape + (EMBEDDING_DIM,))


# speedup vs baseline: 1.8423x; 1.8423x over previous
import functools

import jax
import jax.numpy as jnp
from jax import lax
from jax.experimental import pallas as pl
from jax.experimental.pallas import tpu as pltpu
from jax.experimental.pallas import tpu_sc as plsc

EMBEDDING_DIM = 64
TOTAL = 16384 * 50
NUM_CORES = 2
NUM_SUBCORES = 16
NUM_WORKERS = NUM_CORES * NUM_SUBCORES
PER_WORKER = TOTAL // NUM_WORKERS
CHUNK = 1024
NUM_CHUNKS = PER_WORKER // CHUNK

_mesh = plsc.VectorSubcoreMesh(core_axis_name="c", subcore_axis_name="s")


@functools.partial(
    pl.kernel,
    mesh=_mesh,
    out_type=jax.ShapeDtypeStruct((TOTAL, EMBEDDING_DIM), jnp.float32),
    scratch_types=[
        pltpu.VMEM((CHUNK,), jnp.int32),
        pltpu.VMEM((CHUNK, EMBEDDING_DIM), jnp.float32),
        pltpu.SemaphoreType.DMA,
    ],
    compiler_params=pltpu.CompilerParams(use_tc_tiling_on_sc=False),
)
def _gather(idx_hbm, table_hbm, out_hbm, idx_v, rows_v, sem):
    wid = lax.axis_index("s") * NUM_CORES + lax.axis_index("c")
    base = wid * PER_WORKER

    def body(c, carry):
        off = base + c * CHUNK
        pltpu.sync_copy(idx_hbm.at[pl.ds(off, CHUNK)], idx_v)
        pltpu.async_copy(table_hbm.at[idx_v], rows_v, sem).wait()
        pltpu.sync_copy(rows_v, out_hbm.at[pl.ds(off, CHUNK)])
        return carry

    lax.fori_loop(0, NUM_CHUNKS, body, 0)


def kernel(token_ids, embeddings):
    flat = token_ids.reshape(-1).astype(jnp.int32)
    out = _gather(flat, embeddings)
    return out.reshape(token_ids.shape + (EMBEDDING_DIM,))


# speedup vs baseline: 1.8769x; 1.0188x over previous
import functools

import jax
import jax.numpy as jnp
from jax import lax
from jax.experimental import pallas as pl
from jax.experimental.pallas import tpu as pltpu
from jax.experimental.pallas import tpu_sc as plsc

EMBEDDING_DIM = 64
TOTAL = 16384 * 50
NUM_CORES = 2
NUM_SUBCORES = 16
NUM_WORKERS = NUM_CORES * NUM_SUBCORES
PER_WORKER = TOTAL // NUM_WORKERS
CHUNK = 512
NUM_CHUNKS = PER_WORKER // CHUNK

_mesh = plsc.VectorSubcoreMesh(core_axis_name="c", subcore_axis_name="s")


@functools.partial(
    pl.kernel,
    mesh=_mesh,
    out_type=jax.ShapeDtypeStruct((TOTAL, EMBEDDING_DIM), jnp.float32),
    scratch_types=[
        pltpu.VMEM((NUM_CHUNKS, CHUNK), jnp.int32),
        pltpu.VMEM((CHUNK, EMBEDDING_DIM), jnp.float32),
        pltpu.VMEM((CHUNK, EMBEDDING_DIM), jnp.float32),
        pltpu.SemaphoreType.DMA,
        pltpu.SemaphoreType.DMA,
        pltpu.SemaphoreType.DMA,
        pltpu.SemaphoreType.DMA,
    ],
    compiler_params=pltpu.CompilerParams(use_tc_tiling_on_sc=False),
)
def _gather(idx_hbm, table_hbm, out_hbm, idx_v, rows0, rows1, g0, g1, s0, s1):
    wid = lax.axis_index("s") * NUM_CORES + lax.axis_index("c")
    base = wid * PER_WORKER
    rows = (rows0, rows1)
    gsem = (g0, g1)
    ssem = (s0, s1)

    pltpu.sync_copy(idx_hbm.at[pl.ds(wid * NUM_CHUNKS, NUM_CHUNKS)], idx_v)

    def start_gather(c, b):
        pltpu.async_copy(table_hbm.at[idx_v.at[c]], rows[b], gsem[b])

    def wait_gather(b):
        pltpu.make_async_copy(table_hbm.at[idx_v.at[0]], rows[b], gsem[b]).wait()

    def start_store(c, b):
        pltpu.async_copy(rows[b], out_hbm.at[pl.ds(base + c * CHUNK, CHUNK)], ssem[b])

    def wait_store(b):
        pltpu.make_async_copy(rows[b], out_hbm.at[pl.ds(base, CHUNK)], ssem[b]).wait()

    start_gather(0, 0)
    start_gather(1, 1)

    def body(k, carry):
        wait_gather(0)
        start_store(k, 0)
        wait_store(0)
        start_gather(k + 2, 0)
        wait_gather(1)
        start_store(k + 1, 1)
        wait_store(1)
        start_gather(k + 3, 1)
        return carry

    lax.fori_loop(0, (NUM_CHUNKS - 2) // 2, lambda i, c: body(2 * i, c), 0, unroll=False)

    wait_gather(0)
    start_store(NUM_CHUNKS - 2, 0)
    wait_gather(1)
    start_store(NUM_CHUNKS - 1, 1)
    wait_store(0)
    wait_store(1)


def kernel(token_ids, embeddings):
    idx2d = token_ids.reshape(TOTAL // CHUNK, CHUNK).astype(jnp.int32)
    out = _gather(idx2d, embeddings)
    return out.reshape(token_ids.shape + (EMBEDDING_DIM,))


# speedup vs baseline: 1.8786x; 1.0009x over previous
import functools

import jax
import jax.numpy as jnp
from jax import lax
from jax.experimental import pallas as pl
from jax.experimental.pallas import tpu as pltpu
from jax.experimental.pallas import tpu_sc as plsc

EMBEDDING_DIM = 64
BATCH = 16384
HIST = 50
NUM_CORES = 2
NUM_SUBCORES = 16
NUM_WORKERS = NUM_CORES * NUM_SUBCORES
BLOCK = BATCH // NUM_WORKERS

_mesh = plsc.VectorSubcoreMesh(core_axis_name="c", subcore_axis_name="s")


@functools.partial(
    pl.kernel,
    mesh=_mesh,
    out_type=jax.ShapeDtypeStruct((BATCH, HIST, EMBEDDING_DIM), jnp.float32),
    scratch_types=[
        pltpu.VMEM((HIST, BLOCK), jnp.int32),
        pltpu.VMEM((BLOCK, EMBEDDING_DIM), jnp.float32),
        pltpu.VMEM((BLOCK, EMBEDDING_DIM), jnp.float32),
        pltpu.SemaphoreType.DMA,
        pltpu.SemaphoreType.DMA,
        pltpu.SemaphoreType.DMA,
        pltpu.SemaphoreType.DMA,
    ],
    compiler_params=pltpu.CompilerParams(use_tc_tiling_on_sc=False),
)
def _gather(idx_hbm, table_hbm, out_hbm, idx_v, rows0, rows1, g0, g1, s0, s1):
    wid = lax.axis_index("s") * NUM_CORES + lax.axis_index("c")
    b0 = wid * BLOCK
    rows = (rows0, rows1)
    gsem = (g0, g1)
    ssem = (s0, s1)

    pltpu.sync_copy(idx_hbm.at[:, pl.ds(b0, BLOCK)], idx_v)

    def start_gather(h, b):
        pltpu.async_copy(table_hbm.at[idx_v.at[h]], rows[b], gsem[b])

    def wait_gather(b):
        pltpu.make_async_copy(table_hbm.at[idx_v.at[0]], rows[b], gsem[b]).wait()

    def start_store(h, b):
        pltpu.async_copy(rows[b], out_hbm.at[pl.ds(b0, BLOCK), h], ssem[b])

    def wait_store(b):
        pltpu.make_async_copy(rows[b], out_hbm.at[pl.ds(b0, BLOCK), 0], ssem[b]).wait()

    start_gather(0, 0)
    start_gather(1, 1)

    def body(k, carry):
        wait_gather(0)
        start_store(k, 0)
        wait_store(0)
        start_gather(k + 2, 0)
        wait_gather(1)
        start_store(k + 1, 1)
        wait_store(1)
        start_gather(k + 3, 1)
        return carry

    lax.fori_loop(0, (HIST - 2) // 2, lambda i, c: body(2 * i, c), 0, unroll=False)

    wait_gather(0)
    start_store(HIST - 2, 0)
    wait_gather(1)
    start_store(HIST - 1, 1)
    wait_store(0)
    wait_store(1)


def kernel(token_ids, embeddings):
    return _gather(token_ids.T, embeddings)


# speedup vs baseline: 2.7310x; 1.4537x over previous
import functools

import jax
import jax.numpy as jnp
from jax import lax
from jax.experimental import pallas as pl
from jax.experimental.pallas import tpu as pltpu
from jax.experimental.pallas import tpu_sc as plsc

EMBEDDING_DIM = 64
PAD_DIM = 128
BATCH = 16384
HIST = 50
HIST_PAD = 56
NUM_CORES = 2
NUM_SUBCORES = 16
NUM_WORKERS = NUM_CORES * NUM_SUBCORES
BLOCK = BATCH // NUM_WORKERS

_mesh = plsc.VectorSubcoreMesh(core_axis_name="c", subcore_axis_name="s")


@functools.partial(
    pl.kernel,
    mesh=_mesh,
    out_type=jax.ShapeDtypeStruct((BATCH, HIST_PAD, PAD_DIM), jnp.float32),
    scratch_types=[
        pltpu.VMEM((HIST, BLOCK), jnp.int32),
        pltpu.VMEM((BLOCK, EMBEDDING_DIM), jnp.float32),
        pltpu.VMEM((BLOCK, EMBEDDING_DIM), jnp.float32),
        pltpu.SemaphoreType.DMA,
        pltpu.SemaphoreType.DMA,
        pltpu.SemaphoreType.DMA,
        pltpu.SemaphoreType.DMA,
    ],
    compiler_params=pltpu.CompilerParams(use_tc_tiling_on_sc=False),
)
def _gather(idx_hbm, table_hbm, out_hbm, idx_v, rows0, rows1, g0, g1, s0, s1):
    wid = lax.axis_index("s") * NUM_CORES + lax.axis_index("c")
    b0 = wid * BLOCK
    rows = (rows0, rows1)
    gsem = (g0, g1)
    ssem = (s0, s1)

    pltpu.sync_copy(idx_hbm.at[:, pl.ds(b0, BLOCK)], idx_v)

    def start_gather(h, b):
        pltpu.async_copy(table_hbm.at[idx_v.at[h]], rows[b], gsem[b])

    def wait_gather(b):
        pltpu.make_async_copy(table_hbm.at[idx_v.at[0]], rows[b], gsem[b]).wait()

    def start_store(h, b):
        pltpu.async_copy(
            rows[b], out_hbm.at[pl.ds(b0, BLOCK), h, pl.ds(0, EMBEDDING_DIM)], ssem[b]
        )

    def wait_store(b):
        pltpu.make_async_copy(
            rows[b], out_hbm.at[pl.ds(b0, BLOCK), 0, pl.ds(0, EMBEDDING_DIM)], ssem[b]
        ).wait()

    start_gather(0, 0)
    start_gather(1, 1)

    def body(k, carry):
        wait_gather(0)
        start_store(k, 0)
        wait_store(0)
        start_gather(k + 2, 0)
        wait_gather(1)
        start_store(k + 1, 1)
        wait_store(1)
        start_gather(k + 3, 1)
        return carry

    lax.fori_loop(0, (HIST - 2) // 2, lambda i, c: body(2 * i, c), 0, unroll=False)

    wait_gather(0)
    start_store(HIST - 2, 0)
    wait_gather(1)
    start_store(HIST - 1, 1)
    wait_store(0)
    wait_store(1)


def kernel(token_ids, embeddings):
    table2 = jnp.pad(embeddings, ((0, 0), (0, PAD_DIM - EMBEDDING_DIM))).reshape(
        2 * 1000000, EMBEDDING_DIM
    )
    idx2 = (token_ids * 2).T
    out = _gather(idx2, table2)
    return out[:, :HIST, :EMBEDDING_DIM]
